# initial kernel scaffold (unmeasured)
import jax
import jax.numpy as jnp
from jax import lax
from jax.experimental import pallas as pl
from jax.experimental.pallas import tpu as pltpu


def kernel(
    x,
):
    def body(*refs):
        pass

    out_shape = jax.ShapeDtypeStruct(..., jnp.float32)
    return pl.pallas_call(body, out_shape=out_shape)(...)


# baseline (device time: 120725 ns/iter reference)
import jax
import jax.numpy as jnp
from jax import lax
from jax.experimental import pallas as pl
from jax.experimental.pallas import tpu as pltpu

M = 4096
HALF = 1024


def kernel(x):
    def body(
        x_hbm,
        out_ref,
        stage_ref,
        send_ref,
        recv_ref,
        local_sem,
        stage_sem,
        send_sem,
        recv_sem,
    ):
        my_x = lax.axis_index("x")
        my_y = lax.axis_index("y")
        my_z = lax.axis_index("z")
        peer_y = 1 - my_y

        barrier = pltpu.get_barrier_semaphore()
        pl.semaphore_signal(
            barrier,
            inc=1,
            device_id=(my_x, peer_y, my_z),
            device_id_type=pl.DeviceIdType.MESH,
        )
        pl.semaphore_wait(barrier, 1)

        peer_cp = pltpu.make_async_copy(
            x_hbm.at[0, :, pl.ds(peer_y * HALF, HALF)], stage_ref, stage_sem
        )
        peer_cp.start()
        local_cp = pltpu.make_async_copy(
            x_hbm.at[0, :, pl.ds(my_y * HALF, HALF)], out_ref, local_sem
        )
        local_cp.start()

        peer_cp.wait()
        send_ref[...] = stage_ref[...].astype(jnp.bfloat16)

        rdma = pltpu.make_async_remote_copy(
            src_ref=send_ref,
            dst_ref=recv_ref,
            send_sem=send_sem,
            recv_sem=recv_sem,
            device_id=(my_x, peer_y, my_z),
            device_id_type=pl.DeviceIdType.MESH,
        )
        rdma.start()

        local_cp.wait()
        rdma.wait()
        out_ref[...] += recv_ref[...].astype(jnp.float32)

    return pl.pallas_call(
        body,
        out_shape=jax.ShapeDtypeStruct((M, HALF), jnp.float32),
        in_specs=[pl.BlockSpec(memory_space=pl.ANY)],
        out_specs=pl.BlockSpec(memory_space=pltpu.VMEM),
        scratch_shapes=[
            pltpu.VMEM((M, HALF), jnp.float32),
            pltpu.VMEM((M, HALF), jnp.bfloat16),
            pltpu.VMEM((M, HALF), jnp.bfloat16),
            pltpu.SemaphoreType.DMA,
            pltpu.SemaphoreType.DMA,
            pltpu.SemaphoreType.DMA,
            pltpu.SemaphoreType.DMA,
        ],
        compiler_params=pltpu.CompilerParams(
            collective_id=0,
            vmem_limit_bytes=60 * 1024 * 1024,
        ),
    )(x)


# device time: 77179 ns/iter; 1.5642x vs baseline; 1.5642x over previous
import jax
import jax.numpy as jnp
from jax import lax
from jax.experimental import pallas as pl
from jax.experimental.pallas import tpu as pltpu

M = 4096
HALF_COLS = 1024
HALF_ROWS = 2048
NC = 8
CR = HALF_ROWS // NC


def kernel(x):
    def body(
        x_hbm,
        out_ref,
        stage_ref,
        ysend_ref,
        recv_direct,
        recv_relay,
        local_sem,
        stage_sems,
        ysend_sems,
        yrecv_sems,
        xsend_sems,
        xrecv_sems,
    ):
        my_x = lax.axis_index("x")
        my_y = lax.axis_index("y")
        my_z = lax.axis_index("z")
        peer_y = 1 - my_y
        peer_x = 1 - my_x

        my_col = my_y * HALF_COLS
        peer_col = peer_y * HALF_COLS
        mine_off = my_x * HALF_ROWS
        other_off = peer_x * HALF_ROWS

        barrier = pltpu.get_barrier_semaphore()
        pl.semaphore_signal(
            barrier, inc=1,
            device_id=(my_x, peer_y, my_z),
            device_id_type=pl.DeviceIdType.MESH,
        )
        pl.semaphore_signal(
            barrier, inc=1,
            device_id=(peer_x, my_y, my_z),
            device_id_type=pl.DeviceIdType.MESH,
        )
        pl.semaphore_wait(barrier, 2)

        local_cp = pltpu.make_async_copy(
            x_hbm.at[0, :, pl.ds(my_col, HALF_COLS)], out_ref, local_sem
        )
        local_cp.start()

        stage_cps = []
        for c in range(NC):
            cp = pltpu.make_async_copy(
                x_hbm.at[0, pl.ds(mine_off + c * CR, CR), pl.ds(peer_col, HALF_COLS)],
                stage_ref.at[pl.ds(c * CR, CR), :],
                stage_sems.at[c],
            )
            cp.start()
            stage_cps.append(cp)

        y_rdmas = []
        for c in range(NC):
            stage_cps[c].wait()
            ysend_ref[c * CR:(c + 1) * CR, :] = stage_ref[
                c * CR:(c + 1) * CR, :
            ].astype(jnp.bfloat16)
            rdma = pltpu.make_async_remote_copy(
                src_ref=ysend_ref.at[pl.ds(c * CR, CR), :],
                dst_ref=recv_direct.at[pl.ds(c * CR, CR), :],
                send_sem=ysend_sems.at[c],
                recv_sem=yrecv_sems.at[c],
                device_id=(my_x, peer_y, my_z),
                device_id_type=pl.DeviceIdType.MESH,
            )
            rdma.start()
            y_rdmas.append(rdma)

        local_cp.wait()

        x_rdmas = []
        for c in range(NC):
            y_rdmas[c].wait_recv()
            rdma = pltpu.make_async_remote_copy(
                src_ref=recv_direct.at[pl.ds(c * CR, CR), :],
                dst_ref=recv_relay.at[pl.ds(c * CR, CR), :],
                send_sem=xsend_sems.at[c],
                recv_sem=xrecv_sems.at[c],
                device_id=(peer_x, my_y, my_z),
                device_id_type=pl.DeviceIdType.MESH,
            )
            rdma.start()
            x_rdmas.append(rdma)
            rows = pl.ds(mine_off + c * CR, CR)
            out_ref[rows, :] += recv_direct[
                c * CR:(c + 1) * CR, :
            ].astype(jnp.float32)

        for c in range(NC):
            x_rdmas[c].wait_recv()
            rows = pl.ds(other_off + c * CR, CR)
            out_ref[rows, :] += recv_relay[
                c * CR:(c + 1) * CR, :
            ].astype(jnp.float32)

        for c in range(NC):
            y_rdmas[c].wait_send()
            x_rdmas[c].wait_send()

    return pl.pallas_call(
        body,
        out_shape=jax.ShapeDtypeStruct((M, HALF_COLS), jnp.float32),
        in_specs=[pl.BlockSpec(memory_space=pl.ANY)],
        out_specs=pl.BlockSpec(memory_space=pltpu.VMEM),
        scratch_shapes=[
            pltpu.VMEM((HALF_ROWS, HALF_COLS), jnp.float32),
            pltpu.VMEM((HALF_ROWS, HALF_COLS), jnp.bfloat16),
            pltpu.VMEM((HALF_ROWS, HALF_COLS), jnp.bfloat16),
            pltpu.VMEM((HALF_ROWS, HALF_COLS), jnp.bfloat16),
            pltpu.SemaphoreType.DMA,
            pltpu.SemaphoreType.DMA((NC,)),
            pltpu.SemaphoreType.DMA((NC,)),
            pltpu.SemaphoreType.DMA((NC,)),
            pltpu.SemaphoreType.DMA((NC,)),
            pltpu.SemaphoreType.DMA((NC,)),
        ],
        compiler_params=pltpu.CompilerParams(
            collective_id=0,
            vmem_limit_bytes=60 * 1024 * 1024,
        ),
    )(x)


# device time: 77001 ns/iter; 1.5678x vs baseline; 1.0023x over previous
import jax
import jax.numpy as jnp
from jax import lax
from jax.experimental import pallas as pl
from jax.experimental.pallas import tpu as pltpu

M = 4096
HALF_COLS = 1024
HALF_ROWS = 2048
NC = 8
CR = HALF_ROWS // NC


def kernel(x):
    def body(
        x_hbm,
        out_ref,
        stage_ref,
        ysend_ref,
        recv_direct,
        recv_relay,
        local_sem,
        stage_sems,
        ysend_sems,
        yrecv_sems,
        xsend_sems,
        xrecv_sems,
    ):
        my_x = lax.axis_index("x")
        my_y = lax.axis_index("y")
        my_z = lax.axis_index("z")
        peer_y = 1 - my_y
        peer_x = 1 - my_x

        my_col = my_y * HALF_COLS
        peer_col = peer_y * HALF_COLS
        mine_off = my_x * HALF_ROWS
        other_off = peer_x * HALF_ROWS

        barrier = pltpu.get_barrier_semaphore()
        pl.semaphore_signal(
            barrier, inc=1,
            device_id=(my_x, peer_y, my_z),
            device_id_type=pl.DeviceIdType.MESH,
        )
        pl.semaphore_signal(
            barrier, inc=1,
            device_id=(peer_x, my_y, my_z),
            device_id_type=pl.DeviceIdType.MESH,
        )
        pl.semaphore_wait(barrier, 2)

        stage_cps = []
        for c in range(NC):
            cp = pltpu.make_async_copy(
                x_hbm.at[0, pl.ds(mine_off + c * CR, CR), pl.ds(peer_col, HALF_COLS)],
                stage_ref.at[pl.ds(c * CR, CR), :],
                stage_sems.at[c],
            )
            cp.start()
            stage_cps.append(cp)

        local_cp = pltpu.make_async_copy(
            x_hbm.at[0, :, pl.ds(my_col, HALF_COLS)], out_ref, local_sem
        )
        local_cp.start()

        y_rdmas = []
        for c in range(NC):
            stage_cps[c].wait()
            ysend_ref[c * CR:(c + 1) * CR, :] = stage_ref[
                c * CR:(c + 1) * CR, :
            ].astype(jnp.bfloat16)
            rdma = pltpu.make_async_remote_copy(
                src_ref=ysend_ref.at[pl.ds(c * CR, CR), :],
                dst_ref=recv_direct.at[pl.ds(c * CR, CR), :],
                send_sem=ysend_sems.at[c],
                recv_sem=yrecv_sems.at[c],
                device_id=(my_x, peer_y, my_z),
                device_id_type=pl.DeviceIdType.MESH,
            )
            rdma.start()
            y_rdmas.append(rdma)

        x_rdmas = []
        for c in range(NC):
            y_rdmas[c].wait_recv()
            rdma = pltpu.make_async_remote_copy(
                src_ref=recv_direct.at[pl.ds(c * CR, CR), :],
                dst_ref=recv_relay.at[pl.ds(c * CR, CR), :],
                send_sem=xsend_sems.at[c],
                recv_sem=xrecv_sems.at[c],
                device_id=(peer_x, my_y, my_z),
                device_id_type=pl.DeviceIdType.MESH,
            )
            rdma.start()
            x_rdmas.append(rdma)

        local_cp.wait()
        out_ref[pl.ds(mine_off, HALF_ROWS), :] += recv_direct[...].astype(
            jnp.float32
        )

        for c in range(NC):
            x_rdmas[c].wait_recv()
            rows = pl.ds(other_off + c * CR, CR)
            out_ref[rows, :] += recv_relay[
                c * CR:(c + 1) * CR, :
            ].astype(jnp.float32)

        for c in range(NC):
            y_rdmas[c].wait_send()
            x_rdmas[c].wait_send()

    return pl.pallas_call(
        body,
        out_shape=jax.ShapeDtypeStruct((M, HALF_COLS), jnp.float32),
        in_specs=[pl.BlockSpec(memory_space=pl.ANY)],
        out_specs=pl.BlockSpec(memory_space=pltpu.VMEM),
        scratch_shapes=[
            pltpu.VMEM((HALF_ROWS, HALF_COLS), jnp.float32),
            pltpu.VMEM((HALF_ROWS, HALF_COLS), jnp.bfloat16),
            pltpu.VMEM((HALF_ROWS, HALF_COLS), jnp.bfloat16),
            pltpu.VMEM((HALF_ROWS, HALF_COLS), jnp.bfloat16),
            pltpu.SemaphoreType.DMA,
            pltpu.SemaphoreType.DMA((NC,)),
            pltpu.SemaphoreType.DMA((NC,)),
            pltpu.SemaphoreType.DMA((NC,)),
            pltpu.SemaphoreType.DMA((NC,)),
            pltpu.SemaphoreType.DMA((NC,)),
        ],
        compiler_params=pltpu.CompilerParams(
            collective_id=0,
            vmem_limit_bytes=60 * 1024 * 1024,
        ),
    )(x)


# device time: 68865 ns/iter; 1.7531x vs baseline; 1.1181x over previous
import jax
import jax.numpy as jnp
from jax import lax
from jax.experimental import pallas as pl
from jax.experimental.pallas import tpu as pltpu

M = 4096
HALF_COLS = 1024
HALF_ROWS = 2048
NC = 8
CR = HALF_ROWS // NC


def kernel(x):
    def body(
        x_hbm,
        out_ref,
        local_ref,
        stage_ref,
        ysend_ref,
        recv_direct,
        recv_relay,
        local_sem,
        stage_sems,
        ysend_sems,
        yrecv_sems,
        xsend_sems,
        xrecv_sems,
    ):
        my_x = lax.axis_index("x")
        my_y = lax.axis_index("y")
        my_z = lax.axis_index("z")
        peer_y = 1 - my_y
        peer_x = 1 - my_x

        my_col = my_y * HALF_COLS
        peer_col = peer_y * HALF_COLS
        mine_off = my_x * HALF_ROWS
        other_off = peer_x * HALF_ROWS

        barrier = pltpu.get_barrier_semaphore()
        pl.semaphore_signal(
            barrier, inc=1,
            device_id=(my_x, peer_y, my_z),
            device_id_type=pl.DeviceIdType.MESH,
        )
        pl.semaphore_signal(
            barrier, inc=1,
            device_id=(peer_x, my_y, my_z),
            device_id_type=pl.DeviceIdType.MESH,
        )
        pl.semaphore_wait(barrier, 2)

        stage_cps = []
        for c in range(NC):
            cp = pltpu.make_async_copy(
                x_hbm.at[0, pl.ds(mine_off + c * CR, CR), pl.ds(peer_col, HALF_COLS)],
                stage_ref.at[pl.ds(c * CR, CR), :],
                stage_sems.at[c],
            )
            cp.start()
            stage_cps.append(cp)

        local_cp = pltpu.make_async_copy(
            x_hbm.at[0, :, pl.ds(my_col, HALF_COLS)], local_ref, local_sem
        )
        local_cp.start()

        y_rdmas = []
        for c in range(NC):
            stage_cps[c].wait()
            ysend_ref[c * CR:(c + 1) * CR, :] = stage_ref[
                c * CR:(c + 1) * CR, :
            ].astype(jnp.bfloat16)
            rdma = pltpu.make_async_remote_copy(
                src_ref=ysend_ref.at[pl.ds(c * CR, CR), :],
                dst_ref=recv_direct.at[pl.ds(c * CR, CR), :],
                send_sem=ysend_sems.at[c],
                recv_sem=yrecv_sems.at[c],
                device_id=(my_x, peer_y, my_z),
                device_id_type=pl.DeviceIdType.MESH,
            )
            rdma.start()
            y_rdmas.append(rdma)

        x_rdmas = []
        for c in range(NC):
            y_rdmas[c].wait_recv()
            rdma = pltpu.make_async_remote_copy(
                src_ref=recv_direct.at[pl.ds(c * CR, CR), :],
                dst_ref=recv_relay.at[pl.ds(c * CR, CR), :],
                send_sem=xsend_sems.at[c],
                recv_sem=xrecv_sems.at[c],
                device_id=(peer_x, my_y, my_z),
                device_id_type=pl.DeviceIdType.MESH,
            )
            rdma.start()
            x_rdmas.append(rdma)

        local_cp.wait()
        out_ref[pl.ds(mine_off, HALF_ROWS), :] = (
            local_ref[pl.ds(mine_off, HALF_ROWS), :]
            + recv_direct[...].astype(jnp.float32)
        ).astype(jnp.bfloat16)

        for c in range(NC):
            x_rdmas[c].wait_recv()
            rows = pl.ds(other_off + c * CR, CR)
            out_ref[rows, :] = (
                local_ref[rows, :]
                + recv_relay[c * CR:(c + 1) * CR, :].astype(jnp.float32)
            ).astype(jnp.bfloat16)

        for c in range(NC):
            y_rdmas[c].wait_send()
            x_rdmas[c].wait_send()

    return pl.pallas_call(
        body,
        out_shape=jax.ShapeDtypeStruct((M, HALF_COLS), jnp.bfloat16),
        in_specs=[pl.BlockSpec(memory_space=pl.ANY)],
        out_specs=pl.BlockSpec(memory_space=pltpu.VMEM),
        scratch_shapes=[
            pltpu.VMEM((M, HALF_COLS), jnp.float32),
            pltpu.VMEM((HALF_ROWS, HALF_COLS), jnp.float32),
            pltpu.VMEM((HALF_ROWS, HALF_COLS), jnp.bfloat16),
            pltpu.VMEM((HALF_ROWS, HALF_COLS), jnp.bfloat16),
            pltpu.VMEM((HALF_ROWS, HALF_COLS), jnp.bfloat16),
            pltpu.SemaphoreType.DMA,
            pltpu.SemaphoreType.DMA((NC,)),
            pltpu.SemaphoreType.DMA((NC,)),
            pltpu.SemaphoreType.DMA((NC,)),
            pltpu.SemaphoreType.DMA((NC,)),
            pltpu.SemaphoreType.DMA((NC,)),
        ],
        compiler_params=pltpu.CompilerParams(
            collective_id=0,
            vmem_limit_bytes=60 * 1024 * 1024,
        ),
    )(x)


# device time: 58797 ns/iter; 2.0533x vs baseline; 1.1712x over previous
import jax
import jax.numpy as jnp
from jax import lax
from jax.experimental import pallas as pl
from jax.experimental.pallas import tpu as pltpu

M = 4096
HALF_COLS = 1024
QR = 1024
NCQ = 4
CR = QR // NCQ
HR = QR // 2


def kernel(x):
    def body(
        x_hbm,
        out_ref,
        local_ref,
        stage_ref,
        ysend_ref,
        ydir_ref,
        xdir_ref,
        zdir_ref,
        dbufA_ref,
        dbufB_ref,
        local_sem,
        stage_sems,
        ysend_sems,
        yrecv_sems,
        xsend_sems,
        xrecv_sems,
        zsend_sems,
        zrecv_sems,
        dAsend_sems,
        dArecv_sems,
        dBsend_sems,
        dBrecv_sems,
    ):
        my_x = lax.axis_index("x")
        my_y = lax.axis_index("y")
        my_z = lax.axis_index("z")
        peer_y = 1 - my_y
        peer_x = 1 - my_x
        zbit = my_z % 2
        pair_z = my_z + 1 - 2 * zbit

        y_dev = (my_x, peer_y, my_z)
        x_dev = (peer_x, my_y, my_z)
        z_dev = (my_x, my_y, pair_z)

        my_col = my_y * HALF_COLS
        peer_col = peer_y * HALF_COLS

        qm_off = (2 * my_x + zbit) * QR
        qx_off = (2 * peer_x + zbit) * QR
        qz_off = (2 * my_x + (1 - zbit)) * QR
        qd_off = (2 * peer_x + (1 - zbit)) * QR

        barrier = pltpu.get_barrier_semaphore()
        for dev in (y_dev, x_dev, z_dev):
            pl.semaphore_signal(
                barrier, inc=1, device_id=dev,
                device_id_type=pl.DeviceIdType.MESH,
            )
        pl.semaphore_wait(barrier, 3)

        stage_cps = []
        for c in range(NCQ):
            cp = pltpu.make_async_copy(
                x_hbm.at[0, pl.ds(qm_off + c * CR, CR), pl.ds(peer_col, HALF_COLS)],
                stage_ref.at[pl.ds(c * CR, CR), :],
                stage_sems.at[c],
            )
            cp.start()
            stage_cps.append(cp)

        local_cp = pltpu.make_async_copy(
            x_hbm.at[0, :, pl.ds(my_col, HALF_COLS)], local_ref, local_sem
        )
        local_cp.start()

        y_rdmas = []
        for c in range(NCQ):
            stage_cps[c].wait()
            ysend_ref[c * CR:(c + 1) * CR, :] = stage_ref[
                c * CR:(c + 1) * CR, :
            ].astype(jnp.bfloat16)
            rdma = pltpu.make_async_remote_copy(
                src_ref=ysend_ref.at[pl.ds(c * CR, CR), :],
                dst_ref=ydir_ref.at[pl.ds(c * CR, CR), :],
                send_sem=ysend_sems.at[c],
                recv_sem=yrecv_sems.at[c],
                device_id=y_dev,
                device_id_type=pl.DeviceIdType.MESH,
            )
            rdma.start()
            y_rdmas.append(rdma)

        x_rdmas = []
        z_rdmas = []
        for c in range(NCQ):
            y_rdmas[c].wait_recv()
            rx = pltpu.make_async_remote_copy(
                src_ref=ydir_ref.at[pl.ds(c * CR, CR), :],
                dst_ref=xdir_ref.at[pl.ds(c * CR, CR), :],
                send_sem=xsend_sems.at[c],
                recv_sem=xrecv_sems.at[c],
                device_id=x_dev,
                device_id_type=pl.DeviceIdType.MESH,
            )
            rx.start()
            x_rdmas.append(rx)
            rz = pltpu.make_async_remote_copy(
                src_ref=ydir_ref.at[pl.ds(c * CR, CR), :],
                dst_ref=zdir_ref.at[pl.ds(c * CR, CR), :],
                send_sem=zsend_sems.at[c],
                recv_sem=zrecv_sems.at[c],
                device_id=z_dev,
                device_id_type=pl.DeviceIdType.MESH,
            )
            rz.start()
            z_rdmas.append(rz)

        dA_rdmas = []
        for c in range(2):
            z_rdmas[c].wait_recv()
            rdma = pltpu.make_async_remote_copy(
                src_ref=zdir_ref.at[pl.ds(c * CR, CR), :],
                dst_ref=dbufA_ref.at[pl.ds(c * CR, CR), :],
                send_sem=dAsend_sems.at[c],
                recv_sem=dArecv_sems.at[c],
                device_id=x_dev,
                device_id_type=pl.DeviceIdType.MESH,
            )
            rdma.start()
            dA_rdmas.append(rdma)
        dB_rdmas = []
        for c in range(2, 4):
            x_rdmas[c].wait_recv()
            rdma = pltpu.make_async_remote_copy(
                src_ref=xdir_ref.at[pl.ds(c * CR, CR), :],
                dst_ref=dbufB_ref.at[pl.ds((c - 2) * CR, CR), :],
                send_sem=dBsend_sems.at[c - 2],
                recv_sem=dBrecv_sems.at[c - 2],
                device_id=z_dev,
                device_id_type=pl.DeviceIdType.MESH,
            )
            rdma.start()
            dB_rdmas.append(rdma)

        local_cp.wait()
        out_ref[pl.ds(qm_off, QR), :] = (
            local_ref[pl.ds(qm_off, QR), :] + ydir_ref[...].astype(jnp.float32)
        ).astype(jnp.bfloat16)

        x_rdmas[0].wait_recv()
        x_rdmas[1].wait_recv()
        out_ref[pl.ds(qx_off, QR), :] = (
            local_ref[pl.ds(qx_off, QR), :] + xdir_ref[...].astype(jnp.float32)
        ).astype(jnp.bfloat16)

        z_rdmas[2].wait_recv()
        z_rdmas[3].wait_recv()
        out_ref[pl.ds(qz_off, QR), :] = (
            local_ref[pl.ds(qz_off, QR), :] + zdir_ref[...].astype(jnp.float32)
        ).astype(jnp.bfloat16)

        dA_rdmas[0].wait_recv()
        dA_rdmas[1].wait_recv()
        out_ref[pl.ds(qd_off, HR), :] = (
            local_ref[pl.ds(qd_off, HR), :] + dbufA_ref[...].astype(jnp.float32)
        ).astype(jnp.bfloat16)

        dB_rdmas[0].wait_recv()
        dB_rdmas[1].wait_recv()
        out_ref[pl.ds(qd_off + HR, HR), :] = (
            local_ref[pl.ds(qd_off + HR, HR), :]
            + dbufB_ref[...].astype(jnp.float32)
        ).astype(jnp.bfloat16)

        for r in y_rdmas + x_rdmas + z_rdmas + dA_rdmas + dB_rdmas:
            r.wait_send()

    return pl.pallas_call(
        body,
        out_shape=jax.ShapeDtypeStruct((M, HALF_COLS), jnp.bfloat16),
        in_specs=[pl.BlockSpec(memory_space=pl.ANY)],
        out_specs=pl.BlockSpec(memory_space=pltpu.VMEM),
        scratch_shapes=[
            pltpu.VMEM((M, HALF_COLS), jnp.float32),
            pltpu.VMEM((QR, HALF_COLS), jnp.float32),
            pltpu.VMEM((QR, HALF_COLS), jnp.bfloat16),
            pltpu.VMEM((QR, HALF_COLS), jnp.bfloat16),
            pltpu.VMEM((QR, HALF_COLS), jnp.bfloat16),
            pltpu.VMEM((QR, HALF_COLS), jnp.bfloat16),
            pltpu.VMEM((HR, HALF_COLS), jnp.bfloat16),
            pltpu.VMEM((HR, HALF_COLS), jnp.bfloat16),
            pltpu.SemaphoreType.DMA,
            pltpu.SemaphoreType.DMA((NCQ,)),
            pltpu.SemaphoreType.DMA((NCQ,)),
            pltpu.SemaphoreType.DMA((NCQ,)),
            pltpu.SemaphoreType.DMA((NCQ,)),
            pltpu.SemaphoreType.DMA((NCQ,)),
            pltpu.SemaphoreType.DMA((NCQ,)),
            pltpu.SemaphoreType.DMA((NCQ,)),
            pltpu.SemaphoreType.DMA((2,)),
            pltpu.SemaphoreType.DMA((2,)),
            pltpu.SemaphoreType.DMA((2,)),
            pltpu.SemaphoreType.DMA((2,)),
        ],
        compiler_params=pltpu.CompilerParams(
            collective_id=0,
            vmem_limit_bytes=60 * 1024 * 1024,
        ),
    )(x)


# device time: 56843 ns/iter; 2.1238x vs baseline; 1.0344x over previous
import jax
import jax.numpy as jnp
from jax import lax
from jax.experimental import pallas as pl
from jax.experimental.pallas import tpu as pltpu

M = 4096
HALF_COLS = 1024
QR = 1024
NCQ = 4
CR = QR // NCQ
HR = QR // 2


def kernel(x):
    def body(
        x_hbm,
        out_ref,
        acc_ref,
        local_ref,
        stage_ref,
        ysend_ref,
        ydir_ref,
        xdir_ref,
        zdir_ref,
        dbufA_ref,
        dbufB_ref,
        local_sem,
        stage_sems,
        ysend_sems,
        yrecv_sems,
        xsend_sems,
        xrecv_sems,
        zsend_sems,
        zrecv_sems,
        dAsend_sems,
        dArecv_sems,
        dBsend_sems,
        dBrecv_sems,
        outcp_sems,
    ):
        my_x = lax.axis_index("x")
        my_y = lax.axis_index("y")
        my_z = lax.axis_index("z")
        peer_y = 1 - my_y
        peer_x = 1 - my_x
        zbit = my_z % 2
        pair_z = my_z + 1 - 2 * zbit

        y_dev = (my_x, peer_y, my_z)
        x_dev = (peer_x, my_y, my_z)
        z_dev = (my_x, my_y, pair_z)

        my_col = my_y * HALF_COLS
        peer_col = peer_y * HALF_COLS

        qm_off = (2 * my_x + zbit) * QR
        qx_off = (2 * peer_x + zbit) * QR
        qz_off = (2 * my_x + (1 - zbit)) * QR
        qd_off = (2 * peer_x + (1 - zbit)) * QR

        barrier = pltpu.get_barrier_semaphore()
        for dev in (y_dev, x_dev, z_dev):
            pl.semaphore_signal(
                barrier, inc=1, device_id=dev,
                device_id_type=pl.DeviceIdType.MESH,
            )
        pl.semaphore_wait(barrier, 3)

        stage_cps = []
        for c in range(NCQ):
            cp = pltpu.make_async_copy(
                x_hbm.at[0, pl.ds(qm_off + c * CR, CR), pl.ds(peer_col, HALF_COLS)],
                stage_ref.at[pl.ds(c * CR, CR), :],
                stage_sems.at[c],
            )
            cp.start()
            stage_cps.append(cp)

        local_cp = pltpu.make_async_copy(
            x_hbm.at[0, :, pl.ds(my_col, HALF_COLS)], local_ref, local_sem
        )
        local_cp.start()

        y_rdmas = []
        for c in range(NCQ):
            stage_cps[c].wait()
            ysend_ref[c * CR:(c + 1) * CR, :] = stage_ref[
                c * CR:(c + 1) * CR, :
            ].astype(jnp.bfloat16)
            rdma = pltpu.make_async_remote_copy(
                src_ref=ysend_ref.at[pl.ds(c * CR, CR), :],
                dst_ref=ydir_ref.at[pl.ds(c * CR, CR), :],
                send_sem=ysend_sems.at[c],
                recv_sem=yrecv_sems.at[c],
                device_id=y_dev,
                device_id_type=pl.DeviceIdType.MESH,
            )
            rdma.start()
            y_rdmas.append(rdma)

        x_rdmas = []
        z_rdmas = []
        for c in range(NCQ):
            y_rdmas[c].wait_recv()
            rx = pltpu.make_async_remote_copy(
                src_ref=ydir_ref.at[pl.ds(c * CR, CR), :],
                dst_ref=xdir_ref.at[pl.ds(c * CR, CR), :],
                send_sem=xsend_sems.at[c],
                recv_sem=xrecv_sems.at[c],
                device_id=x_dev,
                device_id_type=pl.DeviceIdType.MESH,
            )
            rx.start()
            x_rdmas.append(rx)
            rz = pltpu.make_async_remote_copy(
                src_ref=ydir_ref.at[pl.ds(c * CR, CR), :],
                dst_ref=zdir_ref.at[pl.ds(c * CR, CR), :],
                send_sem=zsend_sems.at[c],
                recv_sem=zrecv_sems.at[c],
                device_id=z_dev,
                device_id_type=pl.DeviceIdType.MESH,
            )
            rz.start()
            z_rdmas.append(rz)

        dA_rdmas = []
        for c in range(2):
            z_rdmas[c].wait_recv()
            rdma = pltpu.make_async_remote_copy(
                src_ref=zdir_ref.at[pl.ds(c * CR, CR), :],
                dst_ref=dbufA_ref.at[pl.ds(c * CR, CR), :],
                send_sem=dAsend_sems.at[c],
                recv_sem=dArecv_sems.at[c],
                device_id=x_dev,
                device_id_type=pl.DeviceIdType.MESH,
            )
            rdma.start()
            dA_rdmas.append(rdma)
        dB_rdmas = []
        for c in range(2, 4):
            x_rdmas[c].wait_recv()
            rdma = pltpu.make_async_remote_copy(
                src_ref=xdir_ref.at[pl.ds(c * CR, CR), :],
                dst_ref=dbufB_ref.at[pl.ds((c - 2) * CR, CR), :],
                send_sem=dBsend_sems.at[c - 2],
                recv_sem=dBrecv_sems.at[c - 2],
                device_id=z_dev,
                device_id_type=pl.DeviceIdType.MESH,
            )
            rdma.start()
            dB_rdmas.append(rdma)

        out_cps = []

        def _writeback(off, rows, sem_idx):
            cp = pltpu.make_async_copy(
                acc_ref.at[pl.ds(off, rows), :],
                out_ref.at[pl.ds(off, rows), :],
                outcp_sems.at[sem_idx],
            )
            cp.start()
            out_cps.append(cp)

        local_cp.wait()
        acc_ref[pl.ds(qm_off, QR), :] = (
            local_ref[pl.ds(qm_off, QR), :] + ydir_ref[...].astype(jnp.float32)
        ).astype(jnp.bfloat16)
        _writeback(qm_off, QR, 0)

        x_rdmas[0].wait_recv()
        x_rdmas[1].wait_recv()
        acc_ref[pl.ds(qx_off, QR), :] = (
            local_ref[pl.ds(qx_off, QR), :] + xdir_ref[...].astype(jnp.float32)
        ).astype(jnp.bfloat16)
        _writeback(qx_off, QR, 1)

        z_rdmas[2].wait_recv()
        z_rdmas[3].wait_recv()
        acc_ref[pl.ds(qz_off, QR), :] = (
            local_ref[pl.ds(qz_off, QR), :] + zdir_ref[...].astype(jnp.float32)
        ).astype(jnp.bfloat16)
        _writeback(qz_off, QR, 2)

        dA_rdmas[0].wait_recv()
        dA_rdmas[1].wait_recv()
        acc_ref[pl.ds(qd_off, HR), :] = (
            local_ref[pl.ds(qd_off, HR), :] + dbufA_ref[...].astype(jnp.float32)
        ).astype(jnp.bfloat16)
        _writeback(qd_off, HR, 3)

        dB_rdmas[0].wait_recv()
        dB_rdmas[1].wait_recv()
        acc_ref[pl.ds(qd_off + HR, HR), :] = (
            local_ref[pl.ds(qd_off + HR, HR), :]
            + dbufB_ref[...].astype(jnp.float32)
        ).astype(jnp.bfloat16)
        _writeback(qd_off + HR, HR, 4)

        for r in y_rdmas + x_rdmas + z_rdmas + dA_rdmas + dB_rdmas:
            r.wait_send()
        for cp in out_cps:
            cp.wait()

    return pl.pallas_call(
        body,
        out_shape=jax.ShapeDtypeStruct((M, HALF_COLS), jnp.bfloat16),
        in_specs=[pl.BlockSpec(memory_space=pl.ANY)],
        out_specs=pl.BlockSpec(memory_space=pl.ANY),
        scratch_shapes=[
            pltpu.VMEM((M, HALF_COLS), jnp.bfloat16),
            pltpu.VMEM((M, HALF_COLS), jnp.float32),
            pltpu.VMEM((QR, HALF_COLS), jnp.float32),
            pltpu.VMEM((QR, HALF_COLS), jnp.bfloat16),
            pltpu.VMEM((QR, HALF_COLS), jnp.bfloat16),
            pltpu.VMEM((QR, HALF_COLS), jnp.bfloat16),
            pltpu.VMEM((QR, HALF_COLS), jnp.bfloat16),
            pltpu.VMEM((HR, HALF_COLS), jnp.bfloat16),
            pltpu.VMEM((HR, HALF_COLS), jnp.bfloat16),
            pltpu.SemaphoreType.DMA,
            pltpu.SemaphoreType.DMA((NCQ,)),
            pltpu.SemaphoreType.DMA((NCQ,)),
            pltpu.SemaphoreType.DMA((NCQ,)),
            pltpu.SemaphoreType.DMA((NCQ,)),
            pltpu.SemaphoreType.DMA((NCQ,)),
            pltpu.SemaphoreType.DMA((NCQ,)),
            pltpu.SemaphoreType.DMA((NCQ,)),
            pltpu.SemaphoreType.DMA((2,)),
            pltpu.SemaphoreType.DMA((2,)),
            pltpu.SemaphoreType.DMA((2,)),
            pltpu.SemaphoreType.DMA((2,)),
            pltpu.SemaphoreType.DMA((5,)),
        ],
        compiler_params=pltpu.CompilerParams(
            collective_id=0,
            vmem_limit_bytes=60 * 1024 * 1024,
        ),
    )(x)


# device time: 54262 ns/iter; 2.2249x vs baseline; 1.0476x over previous
import jax
import jax.numpy as jnp
from jax import lax
from jax.experimental import pallas as pl
from jax.experimental.pallas import tpu as pltpu

M = 4096
HALF_COLS = 1024
QR = 1024
NCQ = 8
CR = QR // NCQ
HR = QR // 2
HC = NCQ // 2


def kernel(x):
    def body(
        x_hbm,
        out_ref,
        acc_ref,
        local_ref,
        stage_ref,
        ysend_ref,
        ydir_ref,
        xdir_ref,
        zdir_ref,
        dbufA_ref,
        dbufB_ref,
        local_sem,
        stage_sems,
        ysend_sems,
        yrecv_sems,
        xsend_sems,
        xrecv_sems,
        zsend_sems,
        zrecv_sems,
        dAsend_sems,
        dArecv_sems,
        dBsend_sems,
        dBrecv_sems,
        outcp_sems,
    ):
        my_x = lax.axis_index("x")
        my_y = lax.axis_index("y")
        my_z = lax.axis_index("z")
        peer_y = 1 - my_y
        peer_x = 1 - my_x
        zbit = my_z % 2
        pair_z = my_z + 1 - 2 * zbit

        y_dev = (my_x, peer_y, my_z)
        x_dev = (peer_x, my_y, my_z)
        z_dev = (my_x, my_y, pair_z)

        my_col = my_y * HALF_COLS
        peer_col = peer_y * HALF_COLS

        qm_off = (2 * my_x + zbit) * QR
        qx_off = (2 * peer_x + zbit) * QR
        qz_off = (2 * my_x + (1 - zbit)) * QR
        qd_off = (2 * peer_x + (1 - zbit)) * QR

        barrier = pltpu.get_barrier_semaphore()
        for dev in (y_dev, x_dev, z_dev):
            pl.semaphore_signal(
                barrier, inc=1, device_id=dev,
                device_id_type=pl.DeviceIdType.MESH,
            )
        pl.semaphore_wait(barrier, 3)

        stage_cps = []
        for c in range(NCQ):
            cp = pltpu.make_async_copy(
                x_hbm.at[0, pl.ds(qm_off + c * CR, CR), pl.ds(peer_col, HALF_COLS)],
                stage_ref.at[pl.ds(c * CR, CR), :],
                stage_sems.at[c],
            )
            cp.start()
            stage_cps.append(cp)

        local_cp = pltpu.make_async_copy(
            x_hbm.at[0, :, pl.ds(my_col, HALF_COLS)], local_ref, local_sem
        )
        local_cp.start()

        y_rdmas = []
        for c in range(NCQ):
            stage_cps[c].wait()
            ysend_ref[c * CR:(c + 1) * CR, :] = stage_ref[
                c * CR:(c + 1) * CR, :
            ].astype(jnp.bfloat16)
            rdma = pltpu.make_async_remote_copy(
                src_ref=ysend_ref.at[pl.ds(c * CR, CR), :],
                dst_ref=ydir_ref.at[pl.ds(c * CR, CR), :],
                send_sem=ysend_sems.at[c],
                recv_sem=yrecv_sems.at[c],
                device_id=y_dev,
                device_id_type=pl.DeviceIdType.MESH,
            )
            rdma.start()
            y_rdmas.append(rdma)

        x_rdmas = []
        z_rdmas = []
        for c in range(NCQ):
            y_rdmas[c].wait_recv()
            rx = pltpu.make_async_remote_copy(
                src_ref=ydir_ref.at[pl.ds(c * CR, CR), :],
                dst_ref=xdir_ref.at[pl.ds(c * CR, CR), :],
                send_sem=xsend_sems.at[c],
                recv_sem=xrecv_sems.at[c],
                device_id=x_dev,
                device_id_type=pl.DeviceIdType.MESH,
            )
            rx.start()
            x_rdmas.append(rx)
            rz = pltpu.make_async_remote_copy(
                src_ref=ydir_ref.at[pl.ds(c * CR, CR), :],
                dst_ref=zdir_ref.at[pl.ds(c * CR, CR), :],
                send_sem=zsend_sems.at[c],
                recv_sem=zrecv_sems.at[c],
                device_id=z_dev,
                device_id_type=pl.DeviceIdType.MESH,
            )
            rz.start()
            z_rdmas.append(rz)

        dA_rdmas = []
        for c in range(HC):
            z_rdmas[c].wait_recv()
            rdma = pltpu.make_async_remote_copy(
                src_ref=zdir_ref.at[pl.ds(c * CR, CR), :],
                dst_ref=dbufA_ref.at[pl.ds(c * CR, CR), :],
                send_sem=dAsend_sems.at[c],
                recv_sem=dArecv_sems.at[c],
                device_id=x_dev,
                device_id_type=pl.DeviceIdType.MESH,
            )
            rdma.start()
            dA_rdmas.append(rdma)
        dB_rdmas = []
        for c in range(HC, NCQ):
            x_rdmas[c].wait_recv()
            rdma = pltpu.make_async_remote_copy(
                src_ref=xdir_ref.at[pl.ds(c * CR, CR), :],
                dst_ref=dbufB_ref.at[pl.ds((c - HC) * CR, CR), :],
                send_sem=dBsend_sems.at[c - HC],
                recv_sem=dBrecv_sems.at[c - HC],
                device_id=z_dev,
                device_id_type=pl.DeviceIdType.MESH,
            )
            rdma.start()
            dB_rdmas.append(rdma)

        out_cps = []

        def _writeback(off, rows, sem_idx):
            cp = pltpu.make_async_copy(
                acc_ref.at[pl.ds(off, rows), :],
                out_ref.at[pl.ds(off, rows), :],
                outcp_sems.at[sem_idx],
            )
            cp.start()
            out_cps.append(cp)

        local_cp.wait()
        acc_ref[pl.ds(qm_off, QR), :] = (
            local_ref[pl.ds(qm_off, QR), :] + ydir_ref[...].astype(jnp.float32)
        ).astype(jnp.bfloat16)
        _writeback(qm_off, QR, 0)

        for c in range(HC):
            x_rdmas[c].wait_recv()
        acc_ref[pl.ds(qx_off, QR), :] = (
            local_ref[pl.ds(qx_off, QR), :] + xdir_ref[...].astype(jnp.float32)
        ).astype(jnp.bfloat16)
        _writeback(qx_off, QR, 1)

        for c in range(HC, NCQ):
            z_rdmas[c].wait_recv()
        acc_ref[pl.ds(qz_off, QR), :] = (
            local_ref[pl.ds(qz_off, QR), :] + zdir_ref[...].astype(jnp.float32)
        ).astype(jnp.bfloat16)
        _writeback(qz_off, QR, 2)

        for r in dA_rdmas:
            r.wait_recv()
        acc_ref[pl.ds(qd_off, HR), :] = (
            local_ref[pl.ds(qd_off, HR), :] + dbufA_ref[...].astype(jnp.float32)
        ).astype(jnp.bfloat16)
        _writeback(qd_off, HR, 3)

        for r in dB_rdmas:
            r.wait_recv()
        acc_ref[pl.ds(qd_off + HR, HR), :] = (
            local_ref[pl.ds(qd_off + HR, HR), :]
            + dbufB_ref[...].astype(jnp.float32)
        ).astype(jnp.bfloat16)
        _writeback(qd_off + HR, HR, 4)

        for r in y_rdmas + x_rdmas + z_rdmas + dA_rdmas + dB_rdmas:
            r.wait_send()
        for cp in out_cps:
            cp.wait()

    return pl.pallas_call(
        body,
        out_shape=jax.ShapeDtypeStruct((M, HALF_COLS), jnp.bfloat16),
        in_specs=[pl.BlockSpec(memory_space=pl.ANY)],
        out_specs=pl.BlockSpec(memory_space=pl.ANY),
        scratch_shapes=[
            pltpu.VMEM((M, HALF_COLS), jnp.bfloat16),
            pltpu.VMEM((M, HALF_COLS), jnp.float32),
            pltpu.VMEM((QR, HALF_COLS), jnp.float32),
            pltpu.VMEM((QR, HALF_COLS), jnp.bfloat16),
            pltpu.VMEM((QR, HALF_COLS), jnp.bfloat16),
            pltpu.VMEM((QR, HALF_COLS), jnp.bfloat16),
            pltpu.VMEM((QR, HALF_COLS), jnp.bfloat16),
            pltpu.VMEM((HR, HALF_COLS), jnp.bfloat16),
            pltpu.VMEM((HR, HALF_COLS), jnp.bfloat16),
            pltpu.SemaphoreType.DMA,
            pltpu.SemaphoreType.DMA((NCQ,)),
            pltpu.SemaphoreType.DMA((NCQ,)),
            pltpu.SemaphoreType.DMA((NCQ,)),
            pltpu.SemaphoreType.DMA((NCQ,)),
            pltpu.SemaphoreType.DMA((NCQ,)),
            pltpu.SemaphoreType.DMA((NCQ,)),
            pltpu.SemaphoreType.DMA((NCQ,)),
            pltpu.SemaphoreType.DMA((HC,)),
            pltpu.SemaphoreType.DMA((HC,)),
            pltpu.SemaphoreType.DMA((HC,)),
            pltpu.SemaphoreType.DMA((HC,)),
            pltpu.SemaphoreType.DMA((5,)),
        ],
        compiler_params=pltpu.CompilerParams(
            collective_id=0,
            vmem_limit_bytes=60 * 1024 * 1024,
        ),
    )(x)


# device time: 51230 ns/iter; 2.3565x vs baseline; 1.0592x over previous
import jax
import jax.numpy as jnp
from jax import lax
from jax.experimental import pallas as pl
from jax.experimental.pallas import tpu as pltpu

M = 4096
HALF_COLS = 1024
QR = 1024
NCQ = 8
CR = QR // NCQ
EX = 2
AX = 3
BZ = NCQ - EX - AX
NY = NCQ + EX


def kernel(x):
    def body(
        x_hbm,
        out_ref,
        acc_ref,
        local_ref,
        stage_ref,
        ysend_ref,
        ydir_ref,
        ydx_ref,
        xdir_ref,
        zdir_ref,
        dbufA_ref,
        dbufB_ref,
        local_sem,
        stage_sems,
        ysend_sems,
        yrecv_sems,
        xsend_sems,
        xrecv_sems,
        zsend_sems,
        zrecv_sems,
        dAsend_sems,
        dArecv_sems,
        dBsend_sems,
        dBrecv_sems,
        outcp_sems,
    ):
        my_x = lax.axis_index("x")
        my_y = lax.axis_index("y")
        my_z = lax.axis_index("z")
        peer_y = 1 - my_y
        peer_x = 1 - my_x
        zbit = my_z % 2
        pair_z = my_z + 1 - 2 * zbit

        y_dev = (my_x, peer_y, my_z)
        x_dev = (peer_x, my_y, my_z)
        z_dev = (my_x, my_y, pair_z)

        my_col = my_y * HALF_COLS
        peer_col = peer_y * HALF_COLS

        qm_off = (2 * my_x + zbit) * QR
        qx_off = (2 * peer_x + zbit) * QR
        qz_off = (2 * my_x + (1 - zbit)) * QR
        qd_off = (2 * peer_x + (1 - zbit)) * QR

        barrier = pltpu.get_barrier_semaphore()
        for dev in (y_dev, x_dev, z_dev):
            pl.semaphore_signal(
                barrier, inc=1, device_id=dev,
                device_id_type=pl.DeviceIdType.MESH,
            )
        pl.semaphore_wait(barrier, 3)

        stage_cps = []
        for c in range(NY):
            src_row = qm_off + c * CR if c < NCQ else qd_off + (c - NCQ) * CR
            cp = pltpu.make_async_copy(
                x_hbm.at[0, pl.ds(src_row, CR), pl.ds(peer_col, HALF_COLS)],
                stage_ref.at[pl.ds(c * CR, CR), :],
                stage_sems.at[c],
            )
            cp.start()
            stage_cps.append(cp)

        local_cp = pltpu.make_async_copy(
            x_hbm.at[0, :, pl.ds(my_col, HALF_COLS)], local_ref, local_sem
        )
        local_cp.start()

        y_rdmas = []
        for c in range(NY):
            stage_cps[c].wait()
            ysend_ref[c * CR:(c + 1) * CR, :] = stage_ref[
                c * CR:(c + 1) * CR, :
            ].astype(jnp.bfloat16)
            dst = (
                ydir_ref.at[pl.ds(c * CR, CR), :]
                if c < NCQ
                else ydx_ref.at[pl.ds((c - NCQ) * CR, CR), :]
            )
            rdma = pltpu.make_async_remote_copy(
                src_ref=ysend_ref.at[pl.ds(c * CR, CR), :],
                dst_ref=dst,
                send_sem=ysend_sems.at[c],
                recv_sem=yrecv_sems.at[c],
                device_id=y_dev,
                device_id_type=pl.DeviceIdType.MESH,
            )
            rdma.start()
            y_rdmas.append(rdma)

        x_rdmas = []
        z_rdmas = []
        for c in range(NCQ):
            y_rdmas[c].wait_recv()
            rx = pltpu.make_async_remote_copy(
                src_ref=ydir_ref.at[pl.ds(c * CR, CR), :],
                dst_ref=xdir_ref.at[pl.ds(c * CR, CR), :],
                send_sem=xsend_sems.at[c],
                recv_sem=xrecv_sems.at[c],
                device_id=x_dev,
                device_id_type=pl.DeviceIdType.MESH,
            )
            rx.start()
            x_rdmas.append(rx)
            rz = pltpu.make_async_remote_copy(
                src_ref=ydir_ref.at[pl.ds(c * CR, CR), :],
                dst_ref=zdir_ref.at[pl.ds(c * CR, CR), :],
                send_sem=zsend_sems.at[c],
                recv_sem=zrecv_sems.at[c],
                device_id=z_dev,
                device_id_type=pl.DeviceIdType.MESH,
            )
            rz.start()
            z_rdmas.append(rz)

        dA_rdmas = []
        for i, c in enumerate(range(EX, EX + AX)):
            z_rdmas[c].wait_recv()
            rdma = pltpu.make_async_remote_copy(
                src_ref=zdir_ref.at[pl.ds(c * CR, CR), :],
                dst_ref=dbufA_ref.at[pl.ds(i * CR, CR), :],
                send_sem=dAsend_sems.at[i],
                recv_sem=dArecv_sems.at[i],
                device_id=x_dev,
                device_id_type=pl.DeviceIdType.MESH,
            )
            rdma.start()
            dA_rdmas.append(rdma)
        dB_rdmas = []
        for i, c in enumerate(range(EX + AX, NCQ)):
            x_rdmas[c].wait_recv()
            rdma = pltpu.make_async_remote_copy(
                src_ref=xdir_ref.at[pl.ds(c * CR, CR), :],
                dst_ref=dbufB_ref.at[pl.ds(i * CR, CR), :],
                send_sem=dBsend_sems.at[i],
                recv_sem=dBrecv_sems.at[i],
                device_id=z_dev,
                device_id_type=pl.DeviceIdType.MESH,
            )
            rdma.start()
            dB_rdmas.append(rdma)

        out_cps = []

        def _writeback(off, rows, sem_idx):
            cp = pltpu.make_async_copy(
                acc_ref.at[pl.ds(off, rows), :],
                out_ref.at[pl.ds(off, rows), :],
                outcp_sems.at[sem_idx],
            )
            cp.start()
            out_cps.append(cp)

        local_cp.wait()
        acc_ref[pl.ds(qm_off, QR), :] = (
            local_ref[pl.ds(qm_off, QR), :] + ydir_ref[...].astype(jnp.float32)
        ).astype(jnp.bfloat16)
        _writeback(qm_off, QR, 0)

        for c in range(EX + AX):
            x_rdmas[c].wait_recv()
        acc_ref[pl.ds(qx_off, QR), :] = (
            local_ref[pl.ds(qx_off, QR), :] + xdir_ref[...].astype(jnp.float32)
        ).astype(jnp.bfloat16)
        _writeback(qx_off, QR, 1)

        for c in list(range(EX)) + list(range(EX + AX, NCQ)):
            z_rdmas[c].wait_recv()
        acc_ref[pl.ds(qz_off, QR), :] = (
            local_ref[pl.ds(qz_off, QR), :] + zdir_ref[...].astype(jnp.float32)
        ).astype(jnp.bfloat16)
        _writeback(qz_off, QR, 2)

        for c in range(NCQ, NY):
            y_rdmas[c].wait_recv()
        acc_ref[pl.ds(qd_off, EX * CR), :] = (
            local_ref[pl.ds(qd_off, EX * CR), :]
            + ydx_ref[...].astype(jnp.float32)
        ).astype(jnp.bfloat16)
        _writeback(qd_off, EX * CR, 3)

        for r in dA_rdmas:
            r.wait_recv()
        acc_ref[pl.ds(qd_off + EX * CR, AX * CR), :] = (
            local_ref[pl.ds(qd_off + EX * CR, AX * CR), :]
            + dbufA_ref[...].astype(jnp.float32)
        ).astype(jnp.bfloat16)
        _writeback(qd_off + EX * CR, AX * CR, 4)

        for r in dB_rdmas:
            r.wait_recv()
        acc_ref[pl.ds(qd_off + (EX + AX) * CR, BZ * CR), :] = (
            local_ref[pl.ds(qd_off + (EX + AX) * CR, BZ * CR), :]
            + dbufB_ref[...].astype(jnp.float32)
        ).astype(jnp.bfloat16)
        _writeback(qd_off + (EX + AX) * CR, BZ * CR, 5)

        for r in y_rdmas + x_rdmas + z_rdmas + dA_rdmas + dB_rdmas:
            r.wait_send()
        for cp in out_cps:
            cp.wait()

    return pl.pallas_call(
        body,
        out_shape=jax.ShapeDtypeStruct((M, HALF_COLS), jnp.bfloat16),
        in_specs=[pl.BlockSpec(memory_space=pl.ANY)],
        out_specs=pl.BlockSpec(memory_space=pl.ANY),
        scratch_shapes=[
            pltpu.VMEM((M, HALF_COLS), jnp.bfloat16),
            pltpu.VMEM((M, HALF_COLS), jnp.float32),
            pltpu.VMEM((NY * CR, HALF_COLS), jnp.float32),
            pltpu.VMEM((NY * CR, HALF_COLS), jnp.bfloat16),
            pltpu.VMEM((QR, HALF_COLS), jnp.bfloat16),
            pltpu.VMEM((EX * CR, HALF_COLS), jnp.bfloat16),
            pltpu.VMEM((QR, HALF_COLS), jnp.bfloat16),
            pltpu.VMEM((QR, HALF_COLS), jnp.bfloat16),
            pltpu.VMEM((AX * CR, HALF_COLS), jnp.bfloat16),
            pltpu.VMEM((BZ * CR, HALF_COLS), jnp.bfloat16),
            pltpu.SemaphoreType.DMA,
            pltpu.SemaphoreType.DMA((NY,)),
            pltpu.SemaphoreType.DMA((NY,)),
            pltpu.SemaphoreType.DMA((NY,)),
            pltpu.SemaphoreType.DMA((NCQ,)),
            pltpu.SemaphoreType.DMA((NCQ,)),
            pltpu.SemaphoreType.DMA((NCQ,)),
            pltpu.SemaphoreType.DMA((NCQ,)),
            pltpu.SemaphoreType.DMA((AX,)),
            pltpu.SemaphoreType.DMA((AX,)),
            pltpu.SemaphoreType.DMA((BZ,)),
            pltpu.SemaphoreType.DMA((BZ,)),
            pltpu.SemaphoreType.DMA((6,)),
        ],
        compiler_params=pltpu.CompilerParams(
            collective_id=0,
            vmem_limit_bytes=60 * 1024 * 1024,
        ),
    )(x)


# device time: 51184 ns/iter; 2.3586x vs baseline; 1.0009x over previous
import jax
import jax.numpy as jnp
from jax import lax
from jax.experimental import pallas as pl
from jax.experimental.pallas import tpu as pltpu

M = 4096
HALF_COLS = 1024
QR = 1024
NCQ = 8
CR = QR // NCQ
EX = 2
AX = 3
BZ = NCQ - EX - AX
NY = NCQ + EX
YORDER = (5, 6, 7, 2, 3, 4, 0, 1)


def kernel(x):
    def body(
        x_hbm,
        out_ref,
        acc_ref,
        local_ref,
        stage_ref,
        ysend_ref,
        ydir_ref,
        ydx_ref,
        xdir_ref,
        zdir_ref,
        dbufA_ref,
        dbufB_ref,
        local_sem,
        stage_sems,
        ysend_sems,
        yrecv_sems,
        xsend_sems,
        xrecv_sems,
        zsend_sems,
        zrecv_sems,
        dAsend_sems,
        dArecv_sems,
        dBsend_sems,
        dBrecv_sems,
        outcp_sems,
    ):
        my_x = lax.axis_index("x")
        my_y = lax.axis_index("y")
        my_z = lax.axis_index("z")
        peer_y = 1 - my_y
        peer_x = 1 - my_x
        zbit = my_z % 2
        pair_z = my_z + 1 - 2 * zbit

        y_dev = (my_x, peer_y, my_z)
        x_dev = (peer_x, my_y, my_z)
        z_dev = (my_x, my_y, pair_z)

        my_col = my_y * HALF_COLS
        peer_col = peer_y * HALF_COLS

        qm_off = (2 * my_x + zbit) * QR
        qx_off = (2 * peer_x + zbit) * QR
        qz_off = (2 * my_x + (1 - zbit)) * QR
        qd_off = (2 * peer_x + (1 - zbit)) * QR

        barrier = pltpu.get_barrier_semaphore()
        for dev in (y_dev, x_dev, z_dev):
            pl.semaphore_signal(
                barrier, inc=1, device_id=dev,
                device_id_type=pl.DeviceIdType.MESH,
            )
        pl.semaphore_wait(barrier, 3)

        stage_cps = {}
        for c in list(YORDER) + [NCQ, NCQ + 1]:
            src_row = qm_off + c * CR if c < NCQ else qd_off + (c - NCQ) * CR
            cp = pltpu.make_async_copy(
                x_hbm.at[0, pl.ds(src_row, CR), pl.ds(peer_col, HALF_COLS)],
                stage_ref.at[pl.ds(c * CR, CR), :],
                stage_sems.at[c],
            )
            cp.start()
            stage_cps[c] = cp

        local_cp = pltpu.make_async_copy(
            x_hbm.at[0, :, pl.ds(my_col, HALF_COLS)], local_ref, local_sem
        )
        local_cp.start()

        y_rdmas = {}
        for c in list(YORDER) + [NCQ, NCQ + 1]:
            stage_cps[c].wait()
            ysend_ref[c * CR:(c + 1) * CR, :] = stage_ref[
                c * CR:(c + 1) * CR, :
            ].astype(jnp.bfloat16)
            dst = (
                ydir_ref.at[pl.ds(c * CR, CR), :]
                if c < NCQ
                else ydx_ref.at[pl.ds((c - NCQ) * CR, CR), :]
            )
            rdma = pltpu.make_async_remote_copy(
                src_ref=ysend_ref.at[pl.ds(c * CR, CR), :],
                dst_ref=dst,
                send_sem=ysend_sems.at[c],
                recv_sem=yrecv_sems.at[c],
                device_id=y_dev,
                device_id_type=pl.DeviceIdType.MESH,
            )
            rdma.start()
            y_rdmas[c] = rdma

        x_rdmas = {}
        z_rdmas = {}
        for c in YORDER:
            y_rdmas[c].wait_recv()
            rx = pltpu.make_async_remote_copy(
                src_ref=ydir_ref.at[pl.ds(c * CR, CR), :],
                dst_ref=xdir_ref.at[pl.ds(c * CR, CR), :],
                send_sem=xsend_sems.at[c],
                recv_sem=xrecv_sems.at[c],
                device_id=x_dev,
                device_id_type=pl.DeviceIdType.MESH,
            )
            rx.start()
            x_rdmas[c] = rx
            rz = pltpu.make_async_remote_copy(
                src_ref=ydir_ref.at[pl.ds(c * CR, CR), :],
                dst_ref=zdir_ref.at[pl.ds(c * CR, CR), :],
                send_sem=zsend_sems.at[c],
                recv_sem=zrecv_sems.at[c],
                device_id=z_dev,
                device_id_type=pl.DeviceIdType.MESH,
            )
            rz.start()
            z_rdmas[c] = rz

        dB_rdmas = []
        for i, c in enumerate(range(EX + AX, NCQ)):
            x_rdmas[c].wait_recv()
            rdma = pltpu.make_async_remote_copy(
                src_ref=xdir_ref.at[pl.ds(c * CR, CR), :],
                dst_ref=dbufB_ref.at[pl.ds(i * CR, CR), :],
                send_sem=dBsend_sems.at[i],
                recv_sem=dBrecv_sems.at[i],
                device_id=z_dev,
                device_id_type=pl.DeviceIdType.MESH,
            )
            rdma.start()
            dB_rdmas.append(rdma)
        dA_rdmas = []
        for i, c in enumerate(range(EX, EX + AX)):
            z_rdmas[c].wait_recv()
            rdma = pltpu.make_async_remote_copy(
                src_ref=zdir_ref.at[pl.ds(c * CR, CR), :],
                dst_ref=dbufA_ref.at[pl.ds(i * CR, CR), :],
                send_sem=dAsend_sems.at[i],
                recv_sem=dArecv_sems.at[i],
                device_id=x_dev,
                device_id_type=pl.DeviceIdType.MESH,
            )
            rdma.start()
            dA_rdmas.append(rdma)

        out_cps = []

        def _writeback(off, rows, sem_idx):
            cp = pltpu.make_async_copy(
                acc_ref.at[pl.ds(off, rows), :],
                out_ref.at[pl.ds(off, rows), :],
                outcp_sems.at[sem_idx],
            )
            cp.start()
            out_cps.append(cp)

        local_cp.wait()
        acc_ref[pl.ds(qm_off, QR), :] = (
            local_ref[pl.ds(qm_off, QR), :] + ydir_ref[...].astype(jnp.float32)
        ).astype(jnp.bfloat16)
        _writeback(qm_off, QR, 0)

        for c in range(EX + AX):
            x_rdmas[c].wait_recv()
        acc_ref[pl.ds(qx_off, QR), :] = (
            local_ref[pl.ds(qx_off, QR), :] + xdir_ref[...].astype(jnp.float32)
        ).astype(jnp.bfloat16)
        _writeback(qx_off, QR, 1)

        for c in list(range(EX)) + list(range(EX + AX, NCQ)):
            z_rdmas[c].wait_recv()
        acc_ref[pl.ds(qz_off, QR), :] = (
            local_ref[pl.ds(qz_off, QR), :] + zdir_ref[...].astype(jnp.float32)
        ).astype(jnp.bfloat16)
        _writeback(qz_off, QR, 2)

        for c in (NCQ, NCQ + 1):
            y_rdmas[c].wait_recv()
        acc_ref[pl.ds(qd_off, EX * CR), :] = (
            local_ref[pl.ds(qd_off, EX * CR), :]
            + ydx_ref[...].astype(jnp.float32)
        ).astype(jnp.bfloat16)
        _writeback(qd_off, EX * CR, 3)

        for r in dA_rdmas:
            r.wait_recv()
        acc_ref[pl.ds(qd_off + EX * CR, AX * CR), :] = (
            local_ref[pl.ds(qd_off + EX * CR, AX * CR), :]
            + dbufA_ref[...].astype(jnp.float32)
        ).astype(jnp.bfloat16)
        _writeback(qd_off + EX * CR, AX * CR, 4)

        for r in dB_rdmas:
            r.wait_recv()
        acc_ref[pl.ds(qd_off + (EX + AX) * CR, BZ * CR), :] = (
            local_ref[pl.ds(qd_off + (EX + AX) * CR, BZ * CR), :]
            + dbufB_ref[...].astype(jnp.float32)
        ).astype(jnp.bfloat16)
        _writeback(qd_off + (EX + AX) * CR, BZ * CR, 5)

        for r in (
            list(y_rdmas.values())
            + list(x_rdmas.values())
            + list(z_rdmas.values())
            + dA_rdmas
            + dB_rdmas
        ):
            r.wait_send()
        for cp in out_cps:
            cp.wait()

    return pl.pallas_call(
        body,
        out_shape=jax.ShapeDtypeStruct((M, HALF_COLS), jnp.bfloat16),
        in_specs=[pl.BlockSpec(memory_space=pl.ANY)],
        out_specs=pl.BlockSpec(memory_space=pl.ANY),
        scratch_shapes=[
            pltpu.VMEM((M, HALF_COLS), jnp.bfloat16),
            pltpu.VMEM((M, HALF_COLS), jnp.float32),
            pltpu.VMEM((NY * CR, HALF_COLS), jnp.float32),
            pltpu.VMEM((NY * CR, HALF_COLS), jnp.bfloat16),
            pltpu.VMEM((QR, HALF_COLS), jnp.bfloat16),
            pltpu.VMEM((EX * CR, HALF_COLS), jnp.bfloat16),
            pltpu.VMEM((QR, HALF_COLS), jnp.bfloat16),
            pltpu.VMEM((QR, HALF_COLS), jnp.bfloat16),
            pltpu.VMEM((AX * CR, HALF_COLS), jnp.bfloat16),
            pltpu.VMEM((BZ * CR, HALF_COLS), jnp.bfloat16),
            pltpu.SemaphoreType.DMA,
            pltpu.SemaphoreType.DMA((NY,)),
            pltpu.SemaphoreType.DMA((NY,)),
            pltpu.SemaphoreType.DMA((NY,)),
            pltpu.SemaphoreType.DMA((NCQ,)),
            pltpu.SemaphoreType.DMA((NCQ,)),
            pltpu.SemaphoreType.DMA((NCQ,)),
            pltpu.SemaphoreType.DMA((NCQ,)),
            pltpu.SemaphoreType.DMA((AX,)),
            pltpu.SemaphoreType.DMA((AX,)),
            pltpu.SemaphoreType.DMA((BZ,)),
            pltpu.SemaphoreType.DMA((BZ,)),
            pltpu.SemaphoreType.DMA((6,)),
        ],
        compiler_params=pltpu.CompilerParams(
            collective_id=0,
            vmem_limit_bytes=60 * 1024 * 1024,
        ),
    )(x)
